# baseline (speedup 1.0000x reference)
import functools

import jax
import jax.numpy as jnp
import numpy as np
from jax import lax
from jax.experimental import pallas as pl
from jax.experimental.pallas import tpu as pltpu
from jax.experimental.pallas import tpu_sc as plsc

NLVL = 3
BASE_SCALE = 0.25
RES = 3
K = 128
KNN = 32
M = 4096
NB = 2
NR = NB * M
NW = 32
RPW = NR // NW
G = 2
NG = RPW // G

_F32_MIN = np.float32(-3.0e38)
_F32_MAX = np.float32(3.0e38)


def _corr_body(f1_ref, f2_ref, out_ref, *, scale):
    f1 = f1_ref[0]
    f2 = f2_ref[0]
    out_ref[0] = lax.dot_general(
        f1, f2, (((0,), (0,)), ((), ())),
        preferred_element_type=jnp.float32) * scale


def _corr(fmap1, fmap2):
    b, d, n = fmap1.shape
    m = fmap2.shape[2]
    BN = 512
    scale = 1.0 / float(np.sqrt(d))
    return pl.pallas_call(
        functools.partial(_corr_body, scale=scale),
        grid=(b, n // BN),
        in_specs=[
            pl.BlockSpec((1, d, BN), lambda bi, ni: (bi, 0, ni)),
            pl.BlockSpec((1, d, m), lambda bi, ni: (bi, 0, 0)),
        ],
        out_specs=pl.BlockSpec((1, BN, m), lambda bi, ni: (bi, ni, 0)),
        out_shape=jax.ShapeDtypeStruct((b, n, m), jnp.float32),
    )(fmap1, fmap2)


def _sc_process_row(j, rw, cbuf, kbuf, bbk, bbi, hist, idxb,
                    xb, yb, zb, cxb, cyb, czb,
                    otc, odx, ody, odz):
    iota = lax.iota(jnp.int32, 16)
    jv = jnp.full((16,), j, jnp.int32)

    z16 = jnp.zeros((16,), jnp.int32)

    def zero_hist(h, _):
        for u in range(4):
            hist[pl.ds((h * 4 + u) * 16, 16)] = z16
        return _
    lax.fori_loop(0, 16, zero_hist, None)

    ones16 = jnp.ones((16,), jnp.int32)

    def sweep1(p, _):
        for u in range(2):
            i = p * 2 + u
            c = cbuf[j, pl.ds(i * 16, 16)]
            bi_ = lax.bitcast_convert_type(c, jnp.int32)
            kk = bi_ ^ ((bi_ >> 31) | jnp.int32(-2147483648))
            ku = lax.bitcast_convert_type(kk, jnp.uint32)
            kbuf[pl.ds(i * 16, 16)] = ku
            dd = (ku >> 22).astype(jnp.int32)
            plsc.addupdate_scatter(hist, [dd], ones16)
        return _
    lax.fori_loop(0, 128, sweep1, None)

    def scond(st):
        v, t, found, b1 = st
        return jnp.logical_and(jnp.logical_not(found), v >= 0)

    def sbody(st):
        v, t, found, b1 = st
        h = hist[pl.ds(v * 16, 16)]
        hr = lax.rev(h, (0,))
        sr = plsc.cumsum(hr)
        suf = t + lax.rev(sr, (0,))
        msk = suf >= K
        lane = jnp.max(jnp.where(msk, iota, -1))
        fnd = lane >= 0
        nb1 = v * 16 + lane
        tn = t + jnp.sum(h)
        return (v - 1, tn, jnp.logical_or(found, fnd),
                jnp.where(fnd, nb1, b1))

    _, _, _, b1 = lax.while_loop(
        scond, sbody, (63, jnp.int32(0), False, jnp.int32(0)))

    b1u = b1.astype(jnp.uint32)
    lo_t = b1u << 22
    hi_t = lo_t + jnp.uint32(0x3FFFFF)

    def sweep2(p, st):
        for u in range(2):
            i = p * 2 + u
            ku = kbuf[pl.ds(i * 16, 16)]
            m_ge = ku >= lo_t
            nge = jnp.sum(m_ge.astype(jnp.int32))

            def heavy(st2, ku=ku, m_ge=m_ge, i=i):
                pg, pe = st2
                m_gt = ku > hi_t
                m_eq = jnp.logical_and(m_ge, jnp.logical_not(m_gt))
                gi = m_gt.astype(jnp.int32)
                ei = m_eq.astype(jnp.int32)
                pos_g = pg + plsc.cumsum(gi) - 1
                pos_e = pe + plsc.cumsum(ei) - 1
                m_gt_s = jnp.logical_and(m_gt, pos_g < K)
                c = cbuf[j, pl.ds(i * 16, 16)]
                idxv = i * 16 + iota
                plsc.store_scatter(otc, [jv, pos_g], c, mask=m_gt_s)
                plsc.store_scatter(idxb, [pos_g], idxv, mask=m_gt_s)
                kui = lax.bitcast_convert_type(ku, jnp.int32)
                plsc.store_scatter(bbk, [pos_e], kui, mask=m_eq)
                plsc.store_scatter(bbi, [pos_e], idxv, mask=m_eq)
                return (pg + jnp.sum(gi), pe + jnp.sum(ei))

            st = lax.cond(nge > 0, heavy, lambda st2: st2, st)
        return st

    cg, cb = lax.fori_loop(0, 128, sweep2, (jnp.int32(0), jnp.int32(0)))
    r2 = K - cg
    nv = (cb + 15) // 16

    def count_ge(x):
        def cl(q, acc):
            kk = bbk[pl.ds(q * 16, 16)]
            lk = kk & jnp.int32(0x3FFFFF)
            lanes = (q * 16 + iota) < cb
            ok = jnp.logical_and(lk >= x, lanes)
            return acc + jnp.sum(ok.astype(jnp.int32))
        return lax.fori_loop(0, nv, cl, jnp.int32(0))

    def bs(t, st):
        lo, hi = st
        mid = lo + ((hi - lo + 1) >> 1)
        pred = count_ge(mid) >= r2
        return (jnp.where(pred, mid, lo), jnp.where(pred, hi, mid - 1))

    bstar, _ = lax.fori_loop(0, 22, bs, (jnp.int32(0), jnp.int32(0x3FFFFF)))
    cg2 = count_ge(bstar + 1)
    fill = r2 - cg2

    def sweep3(q, st):
        p3, pf = st
        kki = bbk[pl.ds(q * 16, 16)]
        kk = lax.bitcast_convert_type(kki, jnp.uint32)
        ii = bbi[pl.ds(q * 16, 16)]
        lk = kki & jnp.int32(0x3FFFFF)
        lanes = (q * 16 + iota) < cb
        m2g = jnp.logical_and(lk > bstar, lanes)
        m2e = jnp.logical_and(lk == bstar, lanes)
        g2 = m2g.astype(jnp.int32)
        e2 = m2e.astype(jnp.int32)
        pos_g = cg + p3 + plsc.cumsum(g2) - 1
        rank_e = pf + plsc.cumsum(e2)
        m2e_k = jnp.logical_and(m2e, rank_e <= fill)
        pos_e = cg + cg2 + rank_e - 1
        kpos = kk >= jnp.uint32(0x80000000)
        raw = jnp.where(kpos, kk ^ jnp.uint32(0x80000000), ~kk)
        val = lax.bitcast_convert_type(raw, jnp.float32)
        m2g_s = jnp.logical_and(m2g, pos_g < K)
        m2e_s = jnp.logical_and(m2e_k, pos_e < K)
        plsc.store_scatter(otc, [jv, pos_g], val, mask=m2g_s)
        plsc.store_scatter(idxb, [pos_g], ii, mask=m2g_s)
        plsc.store_scatter(otc, [jv, pos_e], val, mask=m2e_s)
        plsc.store_scatter(idxb, [pos_e], ii, mask=m2e_s)
        return (p3 + jnp.sum(g2), pf + jnp.sum(e2))

    lax.fori_loop(0, nv, sweep3, (jnp.int32(0), jnp.int32(0)))

    rwv = jnp.full((16,), rw, jnp.int32)
    cxv = plsc.load_gather(cxb, [rwv])
    cyv = plsc.load_gather(cyb, [rwv])
    czv = plsc.load_gather(czb, [rwv])

    def gth(q, _):
        ids = idxb[pl.ds(q * 16, 16)]
        odx[j, pl.ds(q * 16, 16)] = plsc.load_gather(xb, [ids]) - cxv
        ody[j, pl.ds(q * 16, 16)] = plsc.load_gather(yb, [ids]) - cyv
        odz[j, pl.ds(q * 16, 16)] = plsc.load_gather(zb, [ids]) - czv
        return _
    lax.fori_loop(0, K // 16, gth, None)


def _sc_topk_call(corr2, xx, xy, xz, cbx, cby, cbz):
    mesh = plsc.VectorSubcoreMesh(core_axis_name="c", subcore_axis_name="s")

    @functools.partial(
        pl.kernel,
        out_type=[jax.ShapeDtypeStruct((NR, K), jnp.float32)] * 4,
        mesh=mesh,
        compiler_params=pltpu.CompilerParams(needs_layout_passes=False),
        scratch_types=[
            pltpu.VMEM((G, M), jnp.float32),
            pltpu.VMEM((G, M), jnp.float32),
            pltpu.VMEM((M,), jnp.uint32),
            pltpu.VMEM((M,), jnp.int32),
            pltpu.VMEM((M,), jnp.int32),
            pltpu.VMEM((1024,), jnp.int32),
            pltpu.VMEM((K,), jnp.int32),
            pltpu.VMEM((M,), jnp.float32),
            pltpu.VMEM((M,), jnp.float32),
            pltpu.VMEM((M,), jnp.float32),
            pltpu.VMEM((RPW,), jnp.float32),
            pltpu.VMEM((RPW,), jnp.float32),
            pltpu.VMEM((RPW,), jnp.float32),
            pltpu.VMEM((G, K), jnp.float32),
            pltpu.VMEM((G, K), jnp.float32),
            pltpu.VMEM((G, K), jnp.float32),
            pltpu.VMEM((G, K), jnp.float32),
            pltpu.SemaphoreType.DMA,
            pltpu.SemaphoreType.DMA,
        ],
    )
    def sc_kernel(corr_hbm, xx_hbm, xy_hbm, xz_hbm, cbx_hbm, cby_hbm,
                  cbz_hbm, tc_hbm, dx_hbm, dy_hbm, dz_hbm,
                  cbuf0, cbuf1, kbuf, bbk, bbi, hist, idxb,
                  xb, yb, zb, cxb, cyb, czb,
                  otc, odx, ody, odz, isem0, isem1):
        wid = lax.axis_index("s") * 2 + lax.axis_index("c")
        row0 = wid * RPW
        bi = lax.div(row0, M)

        pltpu.sync_copy(xx_hbm.at[bi], xb)
        pltpu.sync_copy(xy_hbm.at[bi], yb)
        pltpu.sync_copy(xz_hbm.at[bi], zb)
        pltpu.sync_copy(cbx_hbm.at[pl.ds(row0, RPW)], cxb)
        pltpu.sync_copy(cby_hbm.at[pl.ds(row0, RPW)], cyb)
        pltpu.sync_copy(cbz_hbm.at[pl.ds(row0, RPW)], czb)

        def in_copy(g, buf, sem):
            gc = jnp.minimum(g, NG - 1)
            return pltpu.make_async_copy(
                corr_hbm.at[pl.ds(row0 + gc * G, G)], buf, sem)

        in_copy(0, cbuf0, isem0).start()

        def do_group(g, cbuf):
            def rows(j, _):
                _sc_process_row(j, g * G + j, cbuf, kbuf, bbk, bbi, hist,
                                idxb, xb, yb, zb, cxb, cyb, czb,
                                otc, odx, ody, odz)
                return _
            lax.fori_loop(0, G, rows, None)
            r0 = row0 + g * G
            pltpu.sync_copy(otc, tc_hbm.at[pl.ds(r0, G)])
            pltpu.sync_copy(odx, dx_hbm.at[pl.ds(r0, G)])
            pltpu.sync_copy(ody, dy_hbm.at[pl.ds(r0, G)])
            pltpu.sync_copy(odz, dz_hbm.at[pl.ds(r0, G)])

        def pair(p, _):
            g0 = p * 2
            in_copy(g0 + 1, cbuf1, isem1).start()
            in_copy(g0, cbuf0, isem0).wait()
            do_group(g0, cbuf0)
            in_copy(g0 + 2, cbuf0, isem0).start()
            in_copy(g0 + 1, cbuf1, isem1).wait()
            do_group(g0 + 1, cbuf1)
            return _

        lax.fori_loop(0, NG // 2, pair, None)
        in_copy(NG - 1, cbuf0, isem0).wait()

    return sc_kernel(corr2, xx, xy, xz, cbx, cby, cbz)


_RNE_C = np.float32(12582912.0)


def _rne(x):
    return (x + _RNE_C) - _RNE_C


def _tail_body(tc_ref, dx_ref, dy_ref, dz_ref,
               w1t_ref, b1_ref, g1_ref, be1_ref, a1_ref,
               w2t_ref, b2_ref, wk_ref, bk_ref, gk_ref, bek_ref, ak_ref,
               wot_ref, bo_ref, out_ref):
    tc = tc_ref[0]
    dx = dx_ref[0]
    dy = dy_ref[0]
    dz = dz_ref[0]
    a1 = a1_ref[0, 0]
    ak = ak_ref[0, 0]
    eps = jnp.float32(1e-5)

    cols = []
    for lvl in range(NLVL):
        inv_r = jnp.float32(1.0 / (BASE_SCALE * (2 ** lvl)))
        rx = jnp.round(dx * inv_r)
        ry = jnp.round(dy * inv_r)
        rz = jnp.round(dz * inv_r)
        ix = [(rx == jnp.float32(c)).astype(jnp.float32) for c in (-1., 0., 1.)]
        iy = [(ry == jnp.float32(c)).astype(jnp.float32) for c in (-1., 0., 1.)]
        iz = [(rz == jnp.float32(c)).astype(jnp.float32) for c in (-1., 0., 1.)]
        for cx in range(3):
            for cy in range(3):
                pxy = ix[cx] * iy[cy]
                for cz in range(3):
                    w = pxy * iz[cz]
                    sadd = jnp.sum(tc * w, axis=1, keepdims=True)
                    scnt = jnp.sum(w, axis=1, keepdims=True)
                    cols.append(sadd / jnp.clip(scnt, 1.0, float(M)))
    vox = jnp.concatenate(cols, axis=1)

    h = lax.dot_general(vox, w1t_ref[...], (((1,), (0,)), ((), ())),
                        preferred_element_type=jnp.float32)
    h = h + b1_ref[...][None, :]
    scale_parts = []
    shift_parts = []
    for g in range(8):
        sl = h[:, g * 16:(g + 1) * 16]
        mu = jnp.mean(sl)
        var = jnp.mean((sl - mu) ** 2)
        inv = lax.rsqrt(var + eps)
        scale_parts.append(jnp.full((1, 16), 1.0, jnp.float32) * inv)
        shift_parts.append(jnp.full((1, 16), 1.0, jnp.float32) * (-mu * inv))
    svec = jnp.concatenate(scale_parts, axis=1)
    tvec = jnp.concatenate(shift_parts, axis=1)
    g1v = g1_ref[...][None, :]
    hn = (h * svec + tvec) * g1v + be1_ref[...][None, :]
    hp = jnp.where(hn > 0, hn, a1 * hn)
    vf = lax.dot_general(hp, w2t_ref[...], (((1,), (0,)), ((), ())),
                         preferred_element_type=jnp.float32)
    vf = vf + b2_ref[...][None, :]

    d2 = dx * dx + dy * dy + dz * dz
    bits = lax.bitcast_convert_type(d2, jnp.int32)

    lo = jnp.zeros((M, 1), jnp.int32)
    hi = jnp.full((M, 1), jnp.int32(0x7F800000))

    def dbs(t, st):
        lo, hi = st
        mid = lo + ((hi - lo) >> 1)
        cle = jnp.sum((bits <= mid).astype(jnp.float32), axis=1, keepdims=True)
        pred = cle >= float(KNN)
        return (jnp.where(pred, lo, mid + 1), jnp.where(pred, mid, hi))

    lo, hi = lax.fori_loop(0, 31, dbs, (lo, hi))
    bst = lo
    m_lt = bits < bst
    m_eq = bits == bst
    cnt_lt = jnp.sum(m_lt.astype(jnp.float32), axis=1, keepdims=True)
    ri = lax.broadcasted_iota(jnp.int32, (K, K), 0)
    ci = lax.broadcasted_iota(jnp.int32, (K, K), 1)
    tri = (ri <= ci).astype(jnp.float32)
    rank_eq = lax.dot_general(m_eq.astype(jnp.float32), tri,
                              (((1,), (0,)), ((), ())),
                              preferred_element_type=jnp.float32)
    sel = jnp.logical_or(m_lt,
                         jnp.logical_and(m_eq, cnt_lt + rank_eq <= float(KNN)))

    self_ = sel.astype(jnp.float32)
    u = (tc, dx, dy, dz)
    s_tot = [jnp.sum(self_ * uu) for uu in u]
    m_tot = {}
    for i_ in range(4):
        for j_ in range(i_, 4):
            m_tot[(i_, j_)] = jnp.sum(self_ * u[i_] * u[j_])

    wmax_cols = []
    wmin_cols = []
    wk_sc = [[wk_ref[o, c] for c in range(4)] for o in range(64)]
    for o in range(64):
        w0, w1, w2, w3 = wk_sc[o]
        wv = tc * w0 + dx * w1 + dy * w2 + dz * w3
        wmax_cols.append(jnp.max(jnp.where(sel, wv, _F32_MIN),
                                 axis=1, keepdims=True))
        wmin_cols.append(jnp.min(jnp.where(sel, wv, _F32_MAX),
                                 axis=1, keepdims=True))
    wmax = jnp.concatenate(wmax_cols, axis=1)
    wmin = jnp.concatenate(wmin_cols, axis=1)
    bkv = bk_ref[...][None, :]
    h2max = wmax + bkv
    h2min = wmin + bkv

    cnt_g = jnp.float32(8 * M * KNN)
    sc_parts = []
    sh_parts = []
    for g in range(8):
        ssum = jnp.float32(0.0)
        ssq = jnp.float32(0.0)
        for o in range(g * 8, (g + 1) * 8):
            w = wk_sc[o]
            bko = bk_ref[o]
            lin = (w[0] * s_tot[0] + w[1] * s_tot[1]
                   + w[2] * s_tot[2] + w[3] * s_tot[3])
            q = jnp.float32(0.0)
            for i_ in range(4):
                for j_ in range(4):
                    mm = m_tot[(i_, j_)] if i_ <= j_ else m_tot[(j_, i_)]
                    q = q + w[i_] * w[j_] * mm
            ssum = ssum + lin + bko * jnp.float32(M * KNN)
            ssq = ssq + q + 2.0 * bko * lin + bko * bko * jnp.float32(M * KNN)
        mu = ssum / cnt_g
        var = ssq / cnt_g - mu * mu
        inv = lax.rsqrt(var + eps)
        sc_parts.append(jnp.full((1, 8), 1.0, jnp.float32) * inv)
        sh_parts.append(jnp.full((1, 8), 1.0, jnp.float32) * (-mu * inv))
    scv = jnp.concatenate(sc_parts, axis=1)
    shv = jnp.concatenate(sh_parts, axis=1)
    gkv = gk_ref[...][None, :]
    av = gkv * scv
    hsel = jnp.where(av >= 0, h2max, h2min)
    h2n = hsel * av + (gkv * shv + bek_ref[...][None, :])
    h2p = jnp.where(h2n > 0, h2n, ak * h2n)
    knn = lax.dot_general(h2p, wot_ref[...], (((1,), (0,)), ((), ())),
                          preferred_element_type=jnp.float32)
    knn = knn + bo_ref[...][None, :]

    out_ref[0] = vf + knn


def _tail(tcr, dxr, dyr, dzr, W1, b1, g1, be1, a1, W2, b2,
          Wk, bk, gk, bek, ak, Wo, bo):
    full = lambda s: pl.BlockSpec(s, lambda bi: tuple(0 for _ in s))
    big = lambda: pl.BlockSpec((1, M, K), lambda bi: (bi, 0, 0))
    return pl.pallas_call(
        _tail_body,
        grid=(NB,),
        in_specs=[
            big(), big(), big(), big(),
            full((81, 128)), full((128,)), full((128,)), full((128,)),
            full((1, 1)),
            full((128, 64)), full((64,)),
            full((64, 4)), full((64,)), full((64,)), full((64,)),
            full((1, 1)),
            full((64, 64)), full((64,)),
        ],
        out_specs=pl.BlockSpec((1, M, 64), lambda bi: (bi, 0, 0)),
        out_shape=jax.ShapeDtypeStruct((NB, M, 64), jnp.float32),
    )(tcr, dxr, dyr, dzr,
      W1.T, b1, g1, be1, a1.reshape(1, 1), W2.T, b2,
      Wk, bk, gk, bek, ak.reshape(1, 1), Wo.T, bo)


def kernel(fmap1, fmap2, xyz2, coords, W1, b1, g1, be1, a1, W2, b2,
           Wk, bk, gk, bek, ak, Wo, bo):
    b, d, n = fmap1.shape
    corr = _corr(fmap1, fmap2)
    corr2 = corr.reshape(NR, M)
    xx = xyz2[:, :, 0]
    xy = xyz2[:, :, 1]
    xz = xyz2[:, :, 2]
    cfl = coords.reshape(NR, 3)
    cbx = cfl[:, 0]
    cby = cfl[:, 1]
    cbz = cfl[:, 2]
    tcr, dxr, dyr, dzr = _sc_topk_call(corr2, xx, xy, xz, cbx, cby, cbz)
    out = _tail(tcr.reshape(NB, M, K), dxr.reshape(NB, M, K),
                dyr.reshape(NB, M, K), dzr.reshape(NB, M, K),
                W1, b1, g1, be1, a1, W2, b2, Wk, bk, gk, bek, ak, Wo, bo)
    return jnp.transpose(out, (0, 2, 1))


# speedup vs baseline: 1.3309x; 1.3309x over previous
import functools

import jax
import jax.numpy as jnp
import numpy as np
from jax import lax
from jax.experimental import pallas as pl
from jax.experimental.pallas import tpu as pltpu
from jax.experimental.pallas import tpu_sc as plsc

NLVL = 3
BASE_SCALE = 0.25
RES = 3
K = 128
KNN = 32
M = 4096
NB = 2
NR = NB * M
NW = 32
RPW = NR // NW
G = 2
NG = RPW // G

_F32_MIN = np.float32(-3.0e38)
_F32_MAX = np.float32(3.0e38)


def _corr_body(f1_ref, f2_ref, out_ref, *, scale):
    f1 = f1_ref[0]
    f2 = f2_ref[0]
    out_ref[0] = lax.dot_general(
        f1, f2, (((0,), (0,)), ((), ())),
        preferred_element_type=jnp.float32) * scale


def _corr(fmap1, fmap2):
    b, d, n = fmap1.shape
    m = fmap2.shape[2]
    BN = 512
    scale = 1.0 / float(np.sqrt(d))
    return pl.pallas_call(
        functools.partial(_corr_body, scale=scale),
        grid=(b, n // BN),
        in_specs=[
            pl.BlockSpec((1, d, BN), lambda bi, ni: (bi, 0, ni)),
            pl.BlockSpec((1, d, m), lambda bi, ni: (bi, 0, 0)),
        ],
        out_specs=pl.BlockSpec((1, BN, m), lambda bi, ni: (bi, ni, 0)),
        out_shape=jax.ShapeDtypeStruct((b, n, m), jnp.float32),
    )(fmap1, fmap2)


def _sc_process_row(j, rw, cbuf, kbuf, bbk, bbi, hist, idxb,
                    xb, yb, zb, cxb, cyb, czb,
                    otc, odx, ody, odz):
    iota = lax.iota(jnp.int32, 16)
    jv = jnp.full((16,), j, jnp.int32)

    def zero_hist(h, _):
        hist[pl.ds(h * 16, 16)] = jnp.zeros((16,), jnp.int32)
        return _
    lax.fori_loop(0, 64, zero_hist, None)

    def sweep1(i, _):
        c = cbuf[j, pl.ds(i * 16, 16)]
        bi_ = lax.bitcast_convert_type(c, jnp.int32)
        kk = bi_ ^ ((bi_ >> 31) | jnp.int32(-2147483648))
        ku = lax.bitcast_convert_type(kk, jnp.uint32)
        kbuf[pl.ds(i * 16, 16)] = ku
        dd = (ku >> 22).astype(jnp.int32)
        plsc.addupdate_scatter(hist, [dd], jnp.ones((16,), jnp.int32))
        return _
    lax.fori_loop(0, 256, sweep1, None)

    def scond(st):
        v, t, found, b1 = st
        return jnp.logical_and(jnp.logical_not(found), v >= 0)

    def sbody(st):
        v, t, found, b1 = st
        h = hist[pl.ds(v * 16, 16)]
        hr = lax.rev(h, (0,))
        sr = plsc.cumsum(hr)
        suf = t + lax.rev(sr, (0,))
        msk = suf >= K
        lane = jnp.max(jnp.where(msk, iota, -1))
        fnd = lane >= 0
        nb1 = v * 16 + lane
        tn = t + jnp.sum(h)
        return (v - 1, tn, jnp.logical_or(found, fnd),
                jnp.where(fnd, nb1, b1))

    _, _, _, b1 = lax.while_loop(
        scond, sbody, (63, jnp.int32(0), False, jnp.int32(0)))

    b1u = b1.astype(jnp.uint32)
    lo_t = b1u << 22
    hi_t = lo_t + jnp.uint32(0x3FFFFF)

    def sweep2(i, st):
        pg, pe = st
        ku = kbuf[pl.ds(i * 16, 16)]
        m_gt = ku > hi_t
        m_ge = ku >= lo_t
        m_eq = jnp.logical_and(m_ge, jnp.logical_not(m_gt))
        gi = m_gt.astype(jnp.int32)
        ei = m_eq.astype(jnp.int32)
        pos_g = pg + plsc.cumsum(gi) - 1
        pos_e = pe + plsc.cumsum(ei) - 1
        m_gt_s = jnp.logical_and(m_gt, pos_g < K)
        c = cbuf[j, pl.ds(i * 16, 16)]
        idxv = i * 16 + iota
        plsc.store_scatter(otc, [jv, pos_g], c, mask=m_gt_s)
        plsc.store_scatter(idxb, [pos_g], idxv, mask=m_gt_s)
        kui = lax.bitcast_convert_type(ku, jnp.int32)
        plsc.store_scatter(bbk, [pos_e], kui, mask=m_eq)
        plsc.store_scatter(bbi, [pos_e], idxv, mask=m_eq)
        return (pg + jnp.sum(gi), pe + jnp.sum(ei))

    cg, cb = lax.fori_loop(0, 256, sweep2, (jnp.int32(0), jnp.int32(0)))
    r2 = K - cg
    nv = (cb + 15) // 16

    def count_ge(x):
        def cl(q, acc):
            kk = bbk[pl.ds(q * 16, 16)]
            lk = kk & jnp.int32(0x3FFFFF)
            lanes = (q * 16 + iota) < cb
            ok = jnp.logical_and(lk >= x, lanes)
            return acc + jnp.sum(ok.astype(jnp.int32))
        return lax.fori_loop(0, nv, cl, jnp.int32(0))

    def bs(t, st):
        lo, hi = st
        mid = lo + ((hi - lo + 1) >> 1)
        pred = count_ge(mid) >= r2
        return (jnp.where(pred, mid, lo), jnp.where(pred, hi, mid - 1))

    bstar, _ = lax.fori_loop(0, 22, bs, (jnp.int32(0), jnp.int32(0x3FFFFF)))
    cg2 = count_ge(bstar + 1)
    fill = r2 - cg2

    def sweep3(q, st):
        p3, pf = st
        kki = bbk[pl.ds(q * 16, 16)]
        kk = lax.bitcast_convert_type(kki, jnp.uint32)
        ii = bbi[pl.ds(q * 16, 16)]
        lk = kki & jnp.int32(0x3FFFFF)
        lanes = (q * 16 + iota) < cb
        m2g = jnp.logical_and(lk > bstar, lanes)
        m2e = jnp.logical_and(lk == bstar, lanes)
        g2 = m2g.astype(jnp.int32)
        e2 = m2e.astype(jnp.int32)
        pos_g = cg + p3 + plsc.cumsum(g2) - 1
        rank_e = pf + plsc.cumsum(e2)
        m2e_k = jnp.logical_and(m2e, rank_e <= fill)
        pos_e = cg + cg2 + rank_e - 1
        kpos = kk >= jnp.uint32(0x80000000)
        raw = jnp.where(kpos, kk ^ jnp.uint32(0x80000000), ~kk)
        val = lax.bitcast_convert_type(raw, jnp.float32)
        m2g_s = jnp.logical_and(m2g, pos_g < K)
        m2e_s = jnp.logical_and(m2e_k, pos_e < K)
        plsc.store_scatter(otc, [jv, pos_g], val, mask=m2g_s)
        plsc.store_scatter(idxb, [pos_g], ii, mask=m2g_s)
        plsc.store_scatter(otc, [jv, pos_e], val, mask=m2e_s)
        plsc.store_scatter(idxb, [pos_e], ii, mask=m2e_s)
        return (p3 + jnp.sum(g2), pf + jnp.sum(e2))

    lax.fori_loop(0, nv, sweep3, (jnp.int32(0), jnp.int32(0)))

    rwv = jnp.full((16,), rw, jnp.int32)
    cxv = plsc.load_gather(cxb, [rwv])
    cyv = plsc.load_gather(cyb, [rwv])
    czv = plsc.load_gather(czb, [rwv])

    def gth(q, _):
        ids = idxb[pl.ds(q * 16, 16)]
        odx[j, pl.ds(q * 16, 16)] = plsc.load_gather(xb, [ids]) - cxv
        ody[j, pl.ds(q * 16, 16)] = plsc.load_gather(yb, [ids]) - cyv
        odz[j, pl.ds(q * 16, 16)] = plsc.load_gather(zb, [ids]) - czv
        return _
    lax.fori_loop(0, K // 16, gth, None)


def _sc_topk_call(corr2, xx, xy, xz, cbx, cby, cbz):
    mesh = plsc.VectorSubcoreMesh(core_axis_name="c", subcore_axis_name="s")

    @functools.partial(
        pl.kernel,
        out_type=[jax.ShapeDtypeStruct((NR, K), jnp.float32)] * 4,
        mesh=mesh,
        compiler_params=pltpu.CompilerParams(needs_layout_passes=False),
        scratch_types=[
            pltpu.VMEM((G, M), jnp.float32),
            pltpu.VMEM((G, M), jnp.float32),
            pltpu.VMEM((M,), jnp.uint32),
            pltpu.VMEM((M,), jnp.int32),
            pltpu.VMEM((M,), jnp.int32),
            pltpu.VMEM((1024,), jnp.int32),
            pltpu.VMEM((K,), jnp.int32),
            pltpu.VMEM((M,), jnp.float32),
            pltpu.VMEM((M,), jnp.float32),
            pltpu.VMEM((M,), jnp.float32),
            pltpu.VMEM((RPW,), jnp.float32),
            pltpu.VMEM((RPW,), jnp.float32),
            pltpu.VMEM((RPW,), jnp.float32),
            pltpu.VMEM((G, K), jnp.float32),
            pltpu.VMEM((G, K), jnp.float32),
            pltpu.VMEM((G, K), jnp.float32),
            pltpu.VMEM((G, K), jnp.float32),
            pltpu.SemaphoreType.DMA,
            pltpu.SemaphoreType.DMA,
        ],
    )
    def sc_kernel(corr_hbm, xx_hbm, xy_hbm, xz_hbm, cbx_hbm, cby_hbm,
                  cbz_hbm, tc_hbm, dx_hbm, dy_hbm, dz_hbm,
                  cbuf0, cbuf1, kbuf, bbk, bbi, hist, idxb,
                  xb, yb, zb, cxb, cyb, czb,
                  otc, odx, ody, odz, isem0, isem1):
        wid = lax.axis_index("s") * 2 + lax.axis_index("c")
        row0 = wid * RPW
        bi = lax.div(row0, M)

        pltpu.sync_copy(xx_hbm.at[bi], xb)
        pltpu.sync_copy(xy_hbm.at[bi], yb)
        pltpu.sync_copy(xz_hbm.at[bi], zb)
        pltpu.sync_copy(cbx_hbm.at[pl.ds(row0, RPW)], cxb)
        pltpu.sync_copy(cby_hbm.at[pl.ds(row0, RPW)], cyb)
        pltpu.sync_copy(cbz_hbm.at[pl.ds(row0, RPW)], czb)

        def in_copy(g, buf, sem):
            gc = jnp.minimum(g, NG - 1)
            return pltpu.make_async_copy(
                corr_hbm.at[pl.ds(row0 + gc * G, G)], buf, sem)

        in_copy(0, cbuf0, isem0).start()

        def do_group(g, cbuf):
            def rows(j, _):
                _sc_process_row(j, g * G + j, cbuf, kbuf, bbk, bbi, hist,
                                idxb, xb, yb, zb, cxb, cyb, czb,
                                otc, odx, ody, odz)
                return _
            lax.fori_loop(0, G, rows, None)
            r0 = row0 + g * G
            pltpu.sync_copy(otc, tc_hbm.at[pl.ds(r0, G)])
            pltpu.sync_copy(odx, dx_hbm.at[pl.ds(r0, G)])
            pltpu.sync_copy(ody, dy_hbm.at[pl.ds(r0, G)])
            pltpu.sync_copy(odz, dz_hbm.at[pl.ds(r0, G)])

        def pair(p, _):
            g0 = p * 2
            in_copy(g0 + 1, cbuf1, isem1).start()
            in_copy(g0, cbuf0, isem0).wait()
            do_group(g0, cbuf0)
            in_copy(g0 + 2, cbuf0, isem0).start()
            in_copy(g0 + 1, cbuf1, isem1).wait()
            do_group(g0 + 1, cbuf1)
            return _

        lax.fori_loop(0, NG // 2, pair, None)
        in_copy(NG - 1, cbuf0, isem0).wait()

    return sc_kernel(corr2, xx, xy, xz, cbx, cby, cbz)


_RNE_C = np.float32(12582912.0)


def _rne(x):
    return (x + _RNE_C) - _RNE_C


def _tail_body(tc_ref, dx_ref, dy_ref, dz_ref,
               w1t_ref, b1_ref, g1_ref, be1_ref, a1_ref,
               w2t_ref, b2_ref, wk_ref, bk_ref, gk_ref, bek_ref, ak_ref,
               wot_ref, bo_ref, out_ref):
    tc = tc_ref[0]
    dx = dx_ref[0]
    dy = dy_ref[0]
    dz = dz_ref[0]
    a1 = a1_ref[0, 0]
    ak = ak_ref[0, 0]
    eps = jnp.float32(1e-5)

    cols = []
    for lvl in range(NLVL):
        inv_r = jnp.float32(1.0 / (BASE_SCALE * (2 ** lvl)))
        rx = jnp.round(dx * inv_r)
        ry = jnp.round(dy * inv_r)
        rz = jnp.round(dz * inv_r)
        ix = [(rx == jnp.float32(c)).astype(jnp.float32) for c in (-1., 0., 1.)]
        iy = [(ry == jnp.float32(c)).astype(jnp.float32) for c in (-1., 0., 1.)]
        iz = [(rz == jnp.float32(c)).astype(jnp.float32) for c in (-1., 0., 1.)]
        for cx in range(3):
            for cy in range(3):
                pxy = ix[cx] * iy[cy]
                for cz in range(3):
                    w = pxy * iz[cz]
                    sadd = jnp.sum(tc * w, axis=1, keepdims=True)
                    scnt = jnp.sum(w, axis=1, keepdims=True)
                    cols.append(sadd / jnp.clip(scnt, 1.0, float(M)))
    vox = jnp.concatenate(cols, axis=1)

    h = lax.dot_general(vox, w1t_ref[...], (((1,), (0,)), ((), ())),
                        preferred_element_type=jnp.float32)
    h = h + b1_ref[...][None, :]
    scale_parts = []
    shift_parts = []
    for g in range(8):
        sl = h[:, g * 16:(g + 1) * 16]
        mu = jnp.mean(sl)
        var = jnp.mean((sl - mu) ** 2)
        inv = lax.rsqrt(var + eps)
        scale_parts.append(jnp.full((1, 16), 1.0, jnp.float32) * inv)
        shift_parts.append(jnp.full((1, 16), 1.0, jnp.float32) * (-mu * inv))
    svec = jnp.concatenate(scale_parts, axis=1)
    tvec = jnp.concatenate(shift_parts, axis=1)
    g1v = g1_ref[...][None, :]
    hn = (h * svec + tvec) * g1v + be1_ref[...][None, :]
    hp = jnp.where(hn > 0, hn, a1 * hn)
    vf = lax.dot_general(hp, w2t_ref[...], (((1,), (0,)), ((), ())),
                         preferred_element_type=jnp.float32)
    vf = vf + b2_ref[...][None, :]

    d2 = dx * dx + dy * dy + dz * dz
    bits = lax.bitcast_convert_type(d2, jnp.int32)

    lo = jnp.zeros((M, 1), jnp.int32)
    hi = jnp.full((M, 1), jnp.int32(0x7F800000))

    def dbs(t, st):
        lo, hi = st
        mid = lo + ((hi - lo) >> 1)
        cle = jnp.sum((bits <= mid).astype(jnp.float32), axis=1, keepdims=True)
        pred = cle >= float(KNN)
        return (jnp.where(pred, lo, mid + 1), jnp.where(pred, mid, hi))

    lo, hi = lax.fori_loop(0, 31, dbs, (lo, hi))
    bst = lo
    m_lt = bits < bst
    m_eq = bits == bst
    cnt_lt = jnp.sum(m_lt.astype(jnp.float32), axis=1, keepdims=True)
    ri = lax.broadcasted_iota(jnp.int32, (K, K), 0)
    ci = lax.broadcasted_iota(jnp.int32, (K, K), 1)
    tri = (ri <= ci).astype(jnp.float32)
    rank_eq = lax.dot_general(m_eq.astype(jnp.float32), tri,
                              (((1,), (0,)), ((), ())),
                              preferred_element_type=jnp.float32)
    sel = jnp.logical_or(m_lt,
                         jnp.logical_and(m_eq, cnt_lt + rank_eq <= float(KNN)))

    self_ = sel.astype(jnp.float32)
    u = (tc, dx, dy, dz)
    s_tot = [jnp.sum(self_ * uu) for uu in u]
    m_tot = {}
    for i_ in range(4):
        for j_ in range(i_, 4):
            m_tot[(i_, j_)] = jnp.sum(self_ * u[i_] * u[j_])

    wmax_cols = []
    wmin_cols = []
    wk_sc = [[wk_ref[o, c] for c in range(4)] for o in range(64)]
    for o in range(64):
        w0, w1, w2, w3 = wk_sc[o]
        wv = tc * w0 + dx * w1 + dy * w2 + dz * w3
        wmax_cols.append(jnp.max(jnp.where(sel, wv, _F32_MIN),
                                 axis=1, keepdims=True))
        wmin_cols.append(jnp.min(jnp.where(sel, wv, _F32_MAX),
                                 axis=1, keepdims=True))
    wmax = jnp.concatenate(wmax_cols, axis=1)
    wmin = jnp.concatenate(wmin_cols, axis=1)
    bkv = bk_ref[...][None, :]
    h2max = wmax + bkv
    h2min = wmin + bkv

    cnt_g = jnp.float32(8 * M * KNN)
    sc_parts = []
    sh_parts = []
    for g in range(8):
        ssum = jnp.float32(0.0)
        ssq = jnp.float32(0.0)
        for o in range(g * 8, (g + 1) * 8):
            w = wk_sc[o]
            bko = bk_ref[o]
            lin = (w[0] * s_tot[0] + w[1] * s_tot[1]
                   + w[2] * s_tot[2] + w[3] * s_tot[3])
            q = jnp.float32(0.0)
            for i_ in range(4):
                for j_ in range(4):
                    mm = m_tot[(i_, j_)] if i_ <= j_ else m_tot[(j_, i_)]
                    q = q + w[i_] * w[j_] * mm
            ssum = ssum + lin + bko * jnp.float32(M * KNN)
            ssq = ssq + q + 2.0 * bko * lin + bko * bko * jnp.float32(M * KNN)
        mu = ssum / cnt_g
        var = ssq / cnt_g - mu * mu
        inv = lax.rsqrt(var + eps)
        sc_parts.append(jnp.full((1, 8), 1.0, jnp.float32) * inv)
        sh_parts.append(jnp.full((1, 8), 1.0, jnp.float32) * (-mu * inv))
    scv = jnp.concatenate(sc_parts, axis=1)
    shv = jnp.concatenate(sh_parts, axis=1)
    gkv = gk_ref[...][None, :]
    av = gkv * scv
    hsel = jnp.where(av >= 0, h2max, h2min)
    h2n = hsel * av + (gkv * shv + bek_ref[...][None, :])
    h2p = jnp.where(h2n > 0, h2n, ak * h2n)
    knn = lax.dot_general(h2p, wot_ref[...], (((1,), (0,)), ((), ())),
                          preferred_element_type=jnp.float32)
    knn = knn + bo_ref[...][None, :]

    out_ref[0] = vf + knn


def _tail(tcr, dxr, dyr, dzr, W1, b1, g1, be1, a1, W2, b2,
          Wk, bk, gk, bek, ak, Wo, bo):
    full = lambda s: pl.BlockSpec(s, lambda bi: tuple(0 for _ in s))
    big = lambda: pl.BlockSpec((1, M, K), lambda bi: (bi, 0, 0))
    return pl.pallas_call(
        _tail_body,
        grid=(NB,),
        in_specs=[
            big(), big(), big(), big(),
            full((81, 128)), full((128,)), full((128,)), full((128,)),
            full((1, 1)),
            full((128, 64)), full((64,)),
            full((64, 4)), full((64,)), full((64,)), full((64,)),
            full((1, 1)),
            full((64, 64)), full((64,)),
        ],
        out_specs=pl.BlockSpec((1, M, 64), lambda bi: (bi, 0, 0)),
        out_shape=jax.ShapeDtypeStruct((NB, M, 64), jnp.float32),
    )(tcr, dxr, dyr, dzr,
      W1.T, b1, g1, be1, a1.reshape(1, 1), W2.T, b2,
      Wk, bk, gk, bek, ak.reshape(1, 1), Wo.T, bo)


def kernel(fmap1, fmap2, xyz2, coords, W1, b1, g1, be1, a1, W2, b2,
           Wk, bk, gk, bek, ak, Wo, bo):
    b, d, n = fmap1.shape
    corr = _corr(fmap1, fmap2)
    corr2 = corr.reshape(NR, M)
    xx = xyz2[:, :, 0]
    xy = xyz2[:, :, 1]
    xz = xyz2[:, :, 2]
    cfl = coords.reshape(NR, 3)
    cbx = cfl[:, 0]
    cby = cfl[:, 1]
    cbz = cfl[:, 2]
    tcr, dxr, dyr, dzr = _sc_topk_call(corr2, xx, xy, xz, cbx, cby, cbz)
    out = _tail(tcr.reshape(NB, M, K), dxr.reshape(NB, M, K),
                dyr.reshape(NB, M, K), dzr.reshape(NB, M, K),
                W1, b1, g1, be1, a1, W2, b2, Wk, bk, gk, bek, ak, Wo, bo)
    return jnp.transpose(out, (0, 2, 1))
